# baseline (speedup 1.0000x reference)
import functools
import jax
import jax.numpy as jnp
from jax import lax
from jax.experimental import pallas as pl
from jax.experimental.pallas import tpu as pltpu
from jax.experimental.pallas import tpu_sc as plsc

NEI = 32
NEG = float("-inf")
_SCNW = 32


def _sc_gather_call(B, D, CH, bpw, table, gidx):
    mesh = plsc.VectorSubcoreMesh(core_axis_name="c", subcore_axis_name="s")

    @functools.partial(
        pl.kernel, mesh=mesh,
        out_type=jax.ShapeDtypeStruct((B, D), jnp.float32),
        scratch_types=[
            pltpu.VMEM((CH,), jnp.int32),
            pltpu.VMEM((CH, D), jnp.float32),
            pltpu.SemaphoreType.DMA,
        ],
        compiler_params=pltpu.CompilerParams(use_tc_tiling_on_sc=False),
    )
    def k(table_hbm, idx_hbm, out_hbm, idx_v, rows_v, sem):
        wid = lax.axis_index("s") * 2 + lax.axis_index("c")
        base = wid * bpw

        def body(ch, carry):
            off = base + ch * CH
            pltpu.sync_copy(idx_hbm.at[pl.ds(off, CH)], idx_v)
            pltpu.async_copy(table_hbm.at[idx_v], rows_v, sem).wait()
            pltpu.sync_copy(rows_v, out_hbm.at[pl.ds(off, CH)])
            return carry

        lax.fori_loop(0, bpw // CH, body, 0)

    return k(table, gidx)


def _sc_gather(table, gidx, CH):
    R, D = table.shape
    B = gidx.shape[0]
    bpw = B // _SCNW
    return _sc_gather_call(B, D, CH, bpw, table, gidx)


def _knn_body(k, vb, v, x_ref, xr_ref, out_ref):
    X = x_ref[0]
    xr = xr_ref[0]
    inner = lax.dot_general(xr, X, (((1,), (1,)), ((), ())),
                            preferred_element_type=jnp.float32)
    qf = jnp.sum(X * X, axis=1)[None, :]
    qb = jnp.sum(xr * xr, axis=1, keepdims=True)
    neg = 2.0 * inner - qb - qf
    iota = lax.broadcasted_iota(jnp.int32, (vb, v), 1)
    for t in range(k + 1):
        m = jnp.max(neg, axis=1, keepdims=True)
        cand = jnp.where(neg == m, iota, v)
        j = jnp.min(cand, axis=1, keepdims=True)
        if t > 0:
            out_ref[0, :, pl.ds(t - 1, 1)] = j
        neg = jnp.where(iota == j, NEG, neg)


def _knn(vertices, k, vb):
    bs, v, _ = vertices.shape
    return pl.pallas_call(
        functools.partial(_knn_body, k, vb, v),
        grid=(bs, v // vb),
        in_specs=[pl.BlockSpec((1, v, 3), lambda b, i: (b, 0, 0)),
                  pl.BlockSpec((1, vb, 3), lambda b, i: (b, i, 0))],
        out_specs=pl.BlockSpec((1, vb, k), lambda b, i: (b, i, 0)),
        out_shape=jax.ShapeDtypeStruct((bs, v, k), jnp.int32),
    )(vertices, vertices)


def _convc_body(n, vb, c, d, has_sup,
                gg_ref, vr_ref, dir_ref, cen_ref, out_ref):
    vc = vr_ref[0]
    sd = dir_ref[...]
    nrm = jnp.sqrt(jnp.sum(sd * sd, axis=0, keepdims=True))
    sdn = sd / jnp.maximum(nrm, 1e-12)
    acc = jnp.full((vb, c), NEG, jnp.float32)
    for j in range(n):
        gj = gg_ref[0, pl.ds(j * vb, vb)]
        diff = gj[:, :3] - vc
        nn = jnp.sqrt(jnp.sum(diff * diff, axis=1, keepdims=True))
        nd = diff / jnp.maximum(nn, 1e-12)
        th = jnp.maximum(
            jnp.dot(nd, sdn, preferred_element_type=jnp.float32), 0.0)
        act = th * gj[:, 16:16 + c] if has_sup else th
        acc = jnp.maximum(acc, act)
    out_ref[0] = jnp.maximum(cen_ref[0] + acc, 0.0)


def _conv(idxs, verts, support, dirs, center, c, vb, ch):
    bs, v, n = idxs.shape
    d = 16 + c if support is not None else 16
    pad = jnp.zeros((bs, v, 13), jnp.float32)
    parts = [verts, pad] if support is None else [verts, pad, support]
    table = jnp.concatenate(parts, axis=-1).reshape(bs * v, d)
    offs = (jnp.arange(bs, dtype=jnp.int32) * v)[:, None, None, None]
    gidx = (idxs.reshape(bs, v // vb, vb, n).transpose(0, 1, 3, 2)
            + offs).reshape(bs * n * v)
    gg = _sc_gather(table, gidx, ch).reshape(bs * (v // vb), n * vb, d)
    if center is None:
        center = jnp.zeros((bs, v, c), jnp.float32)
    nb = v // vb
    return pl.pallas_call(
        functools.partial(_convc_body, n, vb, c, d, support is not None),
        grid=(bs * nb,),
        in_specs=[pl.BlockSpec((1, n * vb, d), lambda g: (g, 0, 0)),
                  pl.BlockSpec((1, vb, 3), lambda g: (g // nb, g % nb, 0)),
                  pl.BlockSpec((3, c), lambda g: (0, 0)),
                  pl.BlockSpec((1, vb, c), lambda g: (g // nb, g % nb, 0))],
        out_specs=pl.BlockSpec((1, vb, c), lambda g: (g // nb, g % nb, 0)),
        out_shape=jax.ShapeDtypeStruct((bs, v, c), jnp.float32),
    )(gg, verts, dirs, center)


def _mm_body(x_ref, w_ref, b_ref, o_ref):
    o_ref[...] = (jnp.dot(x_ref[...], w_ref[...],
                          preferred_element_type=jnp.float32) + b_ref[...])


def _mm(x, w, b):
    bs, v, cin = x.shape
    cout = w.shape[1]
    out = pl.pallas_call(
        _mm_body,
        out_shape=jax.ShapeDtypeStruct((bs * v, cout), jnp.float32),
    )(x.reshape(bs * v, cin), w, b.reshape(1, cout))
    return out.reshape(bs, v, cout)


def _pool_body(v, np_, c, s_ref, nif_ref, fm_ref, vr_ref, fo_ref, vo_ref):
    s = s_ref[...]
    iota_pv = lax.broadcasted_iota(jnp.int32, (np_, v), 1)
    ohs = (s == iota_pv).astype(jnp.float32)
    hi = lax.Precision.HIGHEST
    niv = jnp.dot(ohs, nif_ref[0], preferred_element_type=jnp.float32,
                  precision=hi).astype(jnp.int32)
    vo_ref[0] = jnp.dot(ohs, vr_ref[0], preferred_element_type=jnp.float32,
                        precision=hi)
    acc = jnp.full((np_, c), NEG, jnp.float32)
    for j in range(4):
        oh = (iota_pv == niv[:, j:j + 1]).astype(jnp.float32)
        acc = jnp.maximum(acc, jnp.dot(oh, fm_ref[0],
                                       preferred_element_type=jnp.float32,
                                       precision=hi))
    fo_ref[0] = acc


def _pool(verts, fm, ni4, sidx):
    bs, v, c = fm.shape
    np_ = sidx.shape[0]
    fo, vo = pl.pallas_call(
        functools.partial(_pool_body, v, np_, c),
        grid=(bs,),
        in_specs=[pl.BlockSpec((np_, 1), lambda b: (0, 0)),
                  pl.BlockSpec((1, v, 4), lambda b: (b, 0, 0)),
                  pl.BlockSpec((1, v, c), lambda b: (b, 0, 0)),
                  pl.BlockSpec((1, v, 3), lambda b: (b, 0, 0))],
        out_specs=[pl.BlockSpec((1, np_, c), lambda b: (b, 0, 0)),
                   pl.BlockSpec((1, np_, 3), lambda b: (b, 0, 0))],
        out_shape=[jax.ShapeDtypeStruct((bs, np_, c), jnp.float32),
                   jax.ShapeDtypeStruct((bs, np_, 3), jnp.float32)],
    )(sidx.reshape(np_, 1).astype(jnp.int32), ni4.astype(jnp.float32),
      fm, verts)
    return vo, fo


def _colmax_body(x_ref, o_ref):
    o_ref[0] = jnp.max(x_ref[0], axis=0, keepdims=True)


def _colmax(x):
    bs, v, c = x.shape
    return pl.pallas_call(
        _colmax_body,
        grid=(bs,),
        in_specs=[pl.BlockSpec((1, v, c), lambda b: (b, 0, 0))],
        out_specs=pl.BlockSpec((1, 1, c), lambda b: (b, 0, 0)),
        out_shape=jax.ShapeDtypeStruct((bs, 1, c), jnp.float32),
    )(x)


def _heads_body(fg_ref, cw1, cb1, cg, cbt, cw2, cb2,
                pw1, pb1, pg, pbt, pw2, pb2, cls_ref, hid_ref):
    fg = fg_ref[...]

    def head(w1, b1, g, bt, w2, b2):
        h = jnp.dot(fg, w1[...], preferred_element_type=jnp.float32) + b1[...]
        mu = jnp.mean(h, axis=0, keepdims=True)
        var = jnp.mean((h - mu) * (h - mu), axis=0, keepdims=True)
        h = g[...] * (h - mu) / jnp.sqrt(var + 1e-5) + bt[...]
        h = jnp.maximum(h, 0.0)
        return jnp.dot(h, w2[...], preferred_element_type=jnp.float32) + b2[...]

    cls_ref[...] = head(cw1, cb1, cg, cbt, cw2, cb2)
    hid = head(pw1, pb1, pg, pbt, pw2, pb2)
    nn = jnp.sqrt(jnp.sum(hid * hid, axis=1, keepdims=True))
    hid_ref[...] = hid / jnp.maximum(nn, 1e-12)


def _heads(fg, p):
    bs = fg.shape[0]
    args = [fg]
    for pre in ("cls", "proj"):
        args += [p[pre + "_w1"], p[pre + "_b1"].reshape(1, -1),
                 p[pre + "_bn_g"].reshape(1, -1),
                 p[pre + "_bn_b"].reshape(1, -1),
                 p[pre + "_w2"], p[pre + "_b2"].reshape(1, -1)]
    return pl.pallas_call(
        _heads_body,
        out_shape=[jax.ShapeDtypeStruct((bs, 3), jnp.float32),
                   jax.ShapeDtypeStruct((bs, 128), jnp.float32)],
    )(*args)


def kernel(vertices, params):
    p = params
    bs, v0, _ = vertices.shape

    ni1 = _knn(vertices, NEI, 256)
    fm0 = _conv(ni1, vertices, None, p["dir0"], None, 16, 256, 1024)

    f1 = _mm(fm0, p["w1"], p["b1"])
    fm1 = _conv(ni1, vertices, f1[:, :, 32:], p["dir1"], f1[:, :, :32],
                32, 256, 512)

    s1 = jax.random.permutation(jax.random.key(101), v0)[: v0 // 4]
    v2, fm1p = _pool(vertices, fm1, ni1[:, :, :4], s1)

    ni2 = _knn(v2, NEI, 512)
    f2 = _mm(fm1p, p["w2"], p["b2"])
    fm2 = _conv(ni2, v2, f2[:, :, 64:], p["dir2"], f2[:, :, :64],
                64, 256, 512)

    m2 = _colmax(fm2)
    fm2c = jnp.concatenate(
        [fm2, jnp.broadcast_to(m2, (bs, 512, 64))], axis=-1)
    f3 = _mm(fm2c, p["w3"], p["b3"])
    fm3 = _conv(ni2, v2, f3[:, :, 256:], p["dir3"], f3[:, :, :256],
                256, 128, 128)

    s2 = jax.random.permutation(jax.random.key(202), 512)[:128]
    v3, fm3p = _pool(v2, fm3, ni2[:, :, :4], s2)

    ni3 = _knn(v3, NEI, 128)
    f4 = _mm(fm3p, p["w4"], p["b4"])
    fm4 = _conv(ni3, v3, f4[:, :, 256:], p["dir4"], f4[:, :, :256],
                256, 128, 128)

    m4 = _colmax(fm4)
    fm4c = jnp.concatenate(
        [fm4, jnp.broadcast_to(m4, (bs, 128, 256))], axis=-1)
    f5 = _mm(fm4c, p["w5"], p["b5"])
    fm5 = _conv(ni3, v3, f5[:, :, 1024:], p["dir5"], f5[:, :, :1024],
                1024, 64, 64)

    fg = _colmax(fm5)[:, 0, :]
    cls, hid = _heads(fg, p)
    return (fg, cls, hid)


# speedup vs baseline: 1.2801x; 1.2801x over previous
import functools
import jax
import jax.numpy as jnp
from jax import lax
from jax.experimental import pallas as pl
from jax.experimental.pallas import tpu as pltpu
from jax.experimental.pallas import tpu_sc as plsc

NEI = 32
NEG = float("-inf")
_SCNW = 32


def _sc_gather_call(B, D, CH, bpw, table, gidx):
    mesh = plsc.VectorSubcoreMesh(core_axis_name="c", subcore_axis_name="s")

    @functools.partial(
        pl.kernel, mesh=mesh,
        out_type=jax.ShapeDtypeStruct((B, D), jnp.float32),
        scratch_types=[
            pltpu.VMEM((CH,), jnp.int32),
            pltpu.VMEM((CH, D), jnp.float32),
            pltpu.SemaphoreType.DMA,
        ],
    )
    def k(table_hbm, idx_hbm, out_hbm, idx_v, rows_v, sem):
        wid = lax.axis_index("s") * 2 + lax.axis_index("c")
        base = wid * bpw

        def body(ch, carry):
            off = base + ch * CH
            pltpu.sync_copy(idx_hbm.at[pl.ds(off, CH)], idx_v)
            pltpu.async_copy(table_hbm.at[idx_v], rows_v, sem).wait()
            pltpu.sync_copy(rows_v, out_hbm.at[pl.ds(off, CH)])
            return carry

        lax.fori_loop(0, bpw // CH, body, 0)

    return k(table, gidx)


def _sc_gather(table, gidx, CH):
    R, D = table.shape
    B = gidx.shape[0]
    bpw = B // _SCNW
    return _sc_gather_call(B, D, CH, bpw, table, gidx)


def _knn_body(k, vb, v, x_ref, xr_ref, out_ref):
    X = x_ref[0]
    xr = xr_ref[0]
    inner = lax.dot_general(xr, X, (((1,), (1,)), ((), ())),
                            preferred_element_type=jnp.float32)
    qf = jnp.sum(X * X, axis=1)[None, :]
    qb = jnp.sum(xr * xr, axis=1, keepdims=True)
    neg = 2.0 * inner - qb - qf
    iota = lax.broadcasted_iota(jnp.int32, (vb, v), 1)
    for t in range(k + 1):
        j = jnp.argmax(neg, axis=1, keepdims=True).astype(jnp.int32)
        if t > 0:
            out_ref[0, :, pl.ds(t - 1, 1)] = j
        neg = jnp.where(iota == j, NEG, neg)


def _knn(vertices, k, vb):
    bs, v, _ = vertices.shape
    return pl.pallas_call(
        functools.partial(_knn_body, k, vb, v),
        grid=(bs, v // vb),
        in_specs=[pl.BlockSpec((1, v, 3), lambda b, i: (b, 0, 0)),
                  pl.BlockSpec((1, vb, 3), lambda b, i: (b, i, 0))],
        out_specs=pl.BlockSpec((1, vb, k), lambda b, i: (b, i, 0)),
        out_shape=jax.ShapeDtypeStruct((bs, v, k), jnp.int32),
    )(vertices, vertices)


def _convc_body(n, vb, c, d, has_sup,
                gg_ref, vr_ref, dir_ref, cen_ref, out_ref):
    vc = vr_ref[0]
    sd = dir_ref[...]
    nrm = jnp.sqrt(jnp.sum(sd * sd, axis=0, keepdims=True))
    sdn = sd / jnp.maximum(nrm, 1e-12)
    acc = jnp.full((vb, c), NEG, jnp.float32)
    for j in range(n):
        gj = gg_ref[0, pl.ds(j * vb, vb)]
        diff = gj[:, :3] - vc
        nn = jnp.sqrt(jnp.sum(diff * diff, axis=1, keepdims=True))
        nd = diff / jnp.maximum(nn, 1e-12)
        th = jnp.maximum(
            jnp.dot(nd, sdn, preferred_element_type=jnp.float32), 0.0)
        act = th * gj[:, 16:16 + c] if has_sup else th
        acc = jnp.maximum(acc, act)
    out_ref[0] = jnp.maximum(cen_ref[0] + acc, 0.0)


def _conv(idxs, verts, support, dirs, center, c, vb, ch):
    bs, v, n = idxs.shape
    dr = 16 + c if support is not None else 16
    d = -(-dr // 128) * 128
    pad = jnp.zeros((bs, v, 13), jnp.float32)
    parts = [verts, pad] if support is None else [verts, pad, support]
    if d > dr:
        parts.append(jnp.zeros((bs, v, d - dr), jnp.float32))
    table = jnp.concatenate(parts, axis=-1).reshape(bs * v, d)
    offs = (jnp.arange(bs, dtype=jnp.int32) * v)[:, None, None, None]
    gidx = (idxs.reshape(bs, v // vb, vb, n).transpose(0, 1, 3, 2)
            + offs).reshape(bs * n * v)
    gg = _sc_gather(table, gidx, ch).reshape(bs * (v // vb), n * vb, d)
    if center is None:
        center = jnp.zeros((bs, v, c), jnp.float32)
    nb = v // vb
    return pl.pallas_call(
        functools.partial(_convc_body, n, vb, c, d, support is not None),
        grid=(bs * nb,),
        in_specs=[pl.BlockSpec((1, n * vb, d), lambda g: (g, 0, 0)),
                  pl.BlockSpec((1, vb, 3), lambda g: (g // nb, g % nb, 0)),
                  pl.BlockSpec((3, c), lambda g: (0, 0)),
                  pl.BlockSpec((1, vb, c), lambda g: (g // nb, g % nb, 0))],
        out_specs=pl.BlockSpec((1, vb, c), lambda g: (g // nb, g % nb, 0)),
        out_shape=jax.ShapeDtypeStruct((bs, v, c), jnp.float32),
    )(gg, verts, dirs, center)


def _mm_body(x_ref, w_ref, b_ref, o_ref):
    o_ref[...] = (jnp.dot(x_ref[...], w_ref[...],
                          preferred_element_type=jnp.float32) + b_ref[...])


def _mm(x, w, b):
    bs, v, cin = x.shape
    cout = w.shape[1]
    out = pl.pallas_call(
        _mm_body,
        out_shape=jax.ShapeDtypeStruct((bs * v, cout), jnp.float32),
    )(x.reshape(bs * v, cin), w, b.reshape(1, cout))
    return out.reshape(bs, v, cout)


def _pool_body(v, np_, c, s_ref, nif_ref, fm_ref, vr_ref, fo_ref, vo_ref):
    s = s_ref[...]
    iota_pv = lax.broadcasted_iota(jnp.int32, (np_, v), 1)
    ohs = (s == iota_pv).astype(jnp.float32)
    hi = lax.Precision.HIGHEST
    niv = jnp.dot(ohs, nif_ref[0], preferred_element_type=jnp.float32,
                  precision=hi).astype(jnp.int32)
    vo_ref[0] = jnp.dot(ohs, vr_ref[0], preferred_element_type=jnp.float32,
                        precision=hi)
    acc = jnp.full((np_, c), NEG, jnp.float32)
    for j in range(4):
        oh = (iota_pv == niv[:, j:j + 1]).astype(jnp.float32)
        acc = jnp.maximum(acc, jnp.dot(oh, fm_ref[0],
                                       preferred_element_type=jnp.float32,
                                       precision=hi))
    fo_ref[0] = acc


def _pool(verts, fm, ni4, sidx):
    bs, v, c = fm.shape
    np_ = sidx.shape[0]
    fo, vo = pl.pallas_call(
        functools.partial(_pool_body, v, np_, c),
        grid=(bs,),
        in_specs=[pl.BlockSpec((np_, 1), lambda b: (0, 0)),
                  pl.BlockSpec((1, v, 4), lambda b: (b, 0, 0)),
                  pl.BlockSpec((1, v, c), lambda b: (b, 0, 0)),
                  pl.BlockSpec((1, v, 3), lambda b: (b, 0, 0))],
        out_specs=[pl.BlockSpec((1, np_, c), lambda b: (b, 0, 0)),
                   pl.BlockSpec((1, np_, 3), lambda b: (b, 0, 0))],
        out_shape=[jax.ShapeDtypeStruct((bs, np_, c), jnp.float32),
                   jax.ShapeDtypeStruct((bs, np_, 3), jnp.float32)],
    )(sidx.reshape(np_, 1).astype(jnp.int32), ni4.astype(jnp.float32),
      fm, verts)
    return vo, fo


def _colmax_body(x_ref, o_ref):
    o_ref[0] = jnp.max(x_ref[0], axis=0, keepdims=True)


def _colmax(x):
    bs, v, c = x.shape
    return pl.pallas_call(
        _colmax_body,
        grid=(bs,),
        in_specs=[pl.BlockSpec((1, v, c), lambda b: (b, 0, 0))],
        out_specs=pl.BlockSpec((1, 1, c), lambda b: (b, 0, 0)),
        out_shape=jax.ShapeDtypeStruct((bs, 1, c), jnp.float32),
    )(x)


def _heads_body(fg_ref, cw1, cb1, cg, cbt, cw2, cb2,
                pw1, pb1, pg, pbt, pw2, pb2, cls_ref, hid_ref):
    fg = fg_ref[...]

    def head(w1, b1, g, bt, w2, b2):
        h = jnp.dot(fg, w1[...], preferred_element_type=jnp.float32) + b1[...]
        mu = jnp.mean(h, axis=0, keepdims=True)
        var = jnp.mean((h - mu) * (h - mu), axis=0, keepdims=True)
        h = g[...] * (h - mu) / jnp.sqrt(var + 1e-5) + bt[...]
        h = jnp.maximum(h, 0.0)
        return jnp.dot(h, w2[...], preferred_element_type=jnp.float32) + b2[...]

    cls_ref[...] = head(cw1, cb1, cg, cbt, cw2, cb2)
    hid = head(pw1, pb1, pg, pbt, pw2, pb2)
    nn = jnp.sqrt(jnp.sum(hid * hid, axis=1, keepdims=True))
    hid_ref[...] = hid / jnp.maximum(nn, 1e-12)


def _heads(fg, p):
    bs = fg.shape[0]
    args = [fg]
    for pre in ("cls", "proj"):
        args += [p[pre + "_w1"], p[pre + "_b1"].reshape(1, -1),
                 p[pre + "_bn_g"].reshape(1, -1),
                 p[pre + "_bn_b"].reshape(1, -1),
                 p[pre + "_w2"], p[pre + "_b2"].reshape(1, -1)]
    return pl.pallas_call(
        _heads_body,
        out_shape=[jax.ShapeDtypeStruct((bs, 3), jnp.float32),
                   jax.ShapeDtypeStruct((bs, 128), jnp.float32)],
    )(*args)


def kernel(vertices, params):
    p = params
    bs, v0, _ = vertices.shape

    ni1 = _knn(vertices, NEI, 256)
    fm0 = _conv(ni1, vertices, None, p["dir0"], None, 16, 256, 512)

    f1 = _mm(fm0, p["w1"], p["b1"])
    fm1 = _conv(ni1, vertices, f1[:, :, 32:], p["dir1"], f1[:, :, :32],
                32, 256, 512)

    s1 = jax.random.permutation(jax.random.key(101), v0)[: v0 // 4]
    v2, fm1p = _pool(vertices, fm1, ni1[:, :, :4], s1)

    ni2 = _knn(v2, NEI, 512)
    f2 = _mm(fm1p, p["w2"], p["b2"])
    fm2 = _conv(ni2, v2, f2[:, :, 64:], p["dir2"], f2[:, :, :64],
                64, 256, 512)

    m2 = _colmax(fm2)
    fm2c = jnp.concatenate(
        [fm2, jnp.broadcast_to(m2, (bs, 512, 64))], axis=-1)
    f3 = _mm(fm2c, p["w3"], p["b3"])
    fm3 = _conv(ni2, v2, f3[:, :, 256:], p["dir3"], f3[:, :, :256],
                256, 128, 128)

    s2 = jax.random.permutation(jax.random.key(202), 512)[:128]
    v3, fm3p = _pool(v2, fm3, ni2[:, :, :4], s2)

    ni3 = _knn(v3, NEI, 128)
    f4 = _mm(fm3p, p["w4"], p["b4"])
    fm4 = _conv(ni3, v3, f4[:, :, 256:], p["dir4"], f4[:, :, :256],
                256, 128, 128)

    m4 = _colmax(fm4)
    fm4c = jnp.concatenate(
        [fm4, jnp.broadcast_to(m4, (bs, 128, 256))], axis=-1)
    f5 = _mm(fm4c, p["w5"], p["b5"])
    fm5 = _conv(ni3, v3, f5[:, :, 1024:], p["dir5"], f5[:, :, :1024],
                1024, 64, 64)

    fg = _colmax(fm5)[:, 0, :]
    cls, hid = _heads(fg, p)
    return (fg, cls, hid)


# speedup vs baseline: 1.3139x; 1.0264x over previous
import functools
import jax
import jax.numpy as jnp
from jax import lax
from jax.experimental import pallas as pl
from jax.experimental.pallas import tpu as pltpu
from jax.experimental.pallas import tpu_sc as plsc

NEI = 32
NEG = float("-inf")
_SCNW = 32


def _sc_gather_call(B, D, CH, bpw, table, gidx):
    mesh = plsc.VectorSubcoreMesh(core_axis_name="c", subcore_axis_name="s")
    nch = bpw // CH

    @functools.partial(
        pl.kernel, mesh=mesh,
        out_type=jax.ShapeDtypeStruct((B, D), jnp.float32),
        scratch_types=[
            pltpu.VMEM((CH,), jnp.int32),
            pltpu.VMEM((CH,), jnp.int32),
            pltpu.VMEM((CH, D), jnp.float32),
            pltpu.VMEM((CH, D), jnp.float32),
            pltpu.SemaphoreType.DMA,
            pltpu.SemaphoreType.DMA,
        ],
    )
    def k(table_hbm, idx_hbm, out_hbm, ia, ib, ra, rb, sa, sb):
        wid = lax.axis_index("s") * 2 + lax.axis_index("c")
        base = wid * bpw

        def fire(iv, rv, sem, off):
            pltpu.sync_copy(idx_hbm.at[pl.ds(off, CH)], iv)
            pltpu.async_copy(table_hbm.at[iv], rv, sem)

        def drain(iv, rv, sem, off):
            pltpu.make_async_copy(table_hbm.at[iv], rv, sem).wait()
            pltpu.sync_copy(rv, out_hbm.at[pl.ds(off, CH)])

        fire(ia, ra, sa, base)

        def body(i, carry):
            off = base + 2 * i * CH
            fire(ib, rb, sb, off + CH)
            drain(ia, ra, sa, off)
            fire(ia, ra, sa, off + 2 * CH)
            drain(ib, rb, sb, off + CH)
            return carry

        lax.fori_loop(0, (nch - 2) // 2, body, 0)
        off = base + (nch - 2) * CH
        fire(ib, rb, sb, off + CH)
        drain(ia, ra, sa, off)
        drain(ib, rb, sb, off + CH)

    return k(table, gidx)


def _sc_gather(table, gidx, CH):
    R, D = table.shape
    B = gidx.shape[0]
    bpw = B // _SCNW
    return _sc_gather_call(B, D, CH, bpw, table, gidx)


def _knn_body(k, vb, v, x_ref, xr_ref, out_ref):
    X = x_ref[0]
    xr = xr_ref[0]
    inner = lax.dot_general(xr, X, (((1,), (1,)), ((), ())),
                            preferred_element_type=jnp.float32)
    qf = jnp.sum(X * X, axis=1)[None, :]
    qb = jnp.sum(xr * xr, axis=1, keepdims=True)
    neg = 2.0 * inner - qb - qf
    iota = lax.broadcasted_iota(jnp.int32, (vb, v), 1)
    for t in range(k + 1):
        j = jnp.argmax(neg, axis=1, keepdims=True).astype(jnp.int32)
        if t > 0:
            out_ref[0, :, pl.ds(t - 1, 1)] = j
        neg = jnp.where(iota == j, NEG, neg)


def _knn(vertices, k, vb):
    bs, v, _ = vertices.shape
    return pl.pallas_call(
        functools.partial(_knn_body, k, vb, v),
        grid=(bs, v // vb),
        in_specs=[pl.BlockSpec((1, v, 3), lambda b, i: (b, 0, 0)),
                  pl.BlockSpec((1, vb, 3), lambda b, i: (b, i, 0))],
        out_specs=pl.BlockSpec((1, vb, k), lambda b, i: (b, i, 0)),
        out_shape=jax.ShapeDtypeStruct((bs, v, k), jnp.int32),
    )(vertices, vertices)


def _convc_body(n, vb, c, d, has_sup,
                gg_ref, vr_ref, dir_ref, cen_ref, out_ref):
    vc = vr_ref[0]
    sd = dir_ref[...]
    nrm = jnp.sqrt(jnp.sum(sd * sd, axis=0, keepdims=True))
    sdn = sd / jnp.maximum(nrm, 1e-12)
    acc = jnp.full((vb, c), NEG, jnp.float32)
    for j in range(n):
        gj = gg_ref[0, pl.ds(j * vb, vb)]
        diff = gj[:, :3] - vc
        nn = jnp.sqrt(jnp.sum(diff * diff, axis=1, keepdims=True))
        nd = diff / jnp.maximum(nn, 1e-12)
        th = jnp.maximum(
            jnp.dot(nd, sdn, preferred_element_type=jnp.float32), 0.0)
        act = th * gj[:, 16:16 + c] if has_sup else th
        acc = jnp.maximum(acc, act)
    out_ref[0] = jnp.maximum(cen_ref[0] + acc, 0.0)


def _conv(idxs, verts, support, dirs, center, c, vb, ch):
    bs, v, n = idxs.shape
    dr = 16 + c if support is not None else 16
    d = -(-dr // 128) * 128
    pad = jnp.zeros((bs, v, 13), jnp.float32)
    parts = [verts, pad] if support is None else [verts, pad, support]
    if d > dr:
        parts.append(jnp.zeros((bs, v, d - dr), jnp.float32))
    table = jnp.concatenate(parts, axis=-1).reshape(bs * v, d)
    offs = (jnp.arange(bs, dtype=jnp.int32) * v)[:, None, None, None]
    gidx = (idxs.reshape(bs, v // vb, vb, n).transpose(0, 1, 3, 2)
            + offs).reshape(bs * n * v)
    gg = _sc_gather(table, gidx, ch).reshape(bs * (v // vb), n * vb, d)
    if center is None:
        center = jnp.zeros((bs, v, c), jnp.float32)
    nb = v // vb
    return pl.pallas_call(
        functools.partial(_convc_body, n, vb, c, d, support is not None),
        grid=(bs * nb,),
        in_specs=[pl.BlockSpec((1, n * vb, d), lambda g: (g, 0, 0)),
                  pl.BlockSpec((1, vb, 3), lambda g: (g // nb, g % nb, 0)),
                  pl.BlockSpec((3, c), lambda g: (0, 0)),
                  pl.BlockSpec((1, vb, c), lambda g: (g // nb, g % nb, 0))],
        out_specs=pl.BlockSpec((1, vb, c), lambda g: (g // nb, g % nb, 0)),
        out_shape=jax.ShapeDtypeStruct((bs, v, c), jnp.float32),
    )(gg, verts, dirs, center)


def _mm_body(x_ref, w_ref, b_ref, o_ref):
    o_ref[...] = (jnp.dot(x_ref[...], w_ref[...],
                          preferred_element_type=jnp.float32) + b_ref[...])


def _mm(x, w, b):
    bs, v, cin = x.shape
    cout = w.shape[1]
    out = pl.pallas_call(
        _mm_body,
        out_shape=jax.ShapeDtypeStruct((bs * v, cout), jnp.float32),
    )(x.reshape(bs * v, cin), w, b.reshape(1, cout))
    return out.reshape(bs, v, cout)


def _pool_body(v, np_, c, s_ref, nif_ref, fm_ref, vr_ref, fo_ref, vo_ref):
    s = s_ref[...]
    iota_pv = lax.broadcasted_iota(jnp.int32, (np_, v), 1)
    ohs = (s == iota_pv).astype(jnp.float32)
    hi = lax.Precision.HIGHEST
    niv = jnp.dot(ohs, nif_ref[0], preferred_element_type=jnp.float32,
                  precision=hi).astype(jnp.int32)
    vo_ref[0] = jnp.dot(ohs, vr_ref[0], preferred_element_type=jnp.float32,
                        precision=hi)
    acc = jnp.full((np_, c), NEG, jnp.float32)
    for j in range(4):
        oh = (iota_pv == niv[:, j:j + 1]).astype(jnp.float32)
        acc = jnp.maximum(acc, jnp.dot(oh, fm_ref[0],
                                       preferred_element_type=jnp.float32,
                                       precision=hi))
    fo_ref[0] = acc


def _pool(verts, fm, ni4, sidx):
    bs, v, c = fm.shape
    np_ = sidx.shape[0]
    fo, vo = pl.pallas_call(
        functools.partial(_pool_body, v, np_, c),
        grid=(bs,),
        in_specs=[pl.BlockSpec((np_, 1), lambda b: (0, 0)),
                  pl.BlockSpec((1, v, 4), lambda b: (b, 0, 0)),
                  pl.BlockSpec((1, v, c), lambda b: (b, 0, 0)),
                  pl.BlockSpec((1, v, 3), lambda b: (b, 0, 0))],
        out_specs=[pl.BlockSpec((1, np_, c), lambda b: (b, 0, 0)),
                   pl.BlockSpec((1, np_, 3), lambda b: (b, 0, 0))],
        out_shape=[jax.ShapeDtypeStruct((bs, np_, c), jnp.float32),
                   jax.ShapeDtypeStruct((bs, np_, 3), jnp.float32)],
    )(sidx.reshape(np_, 1).astype(jnp.int32), ni4.astype(jnp.float32),
      fm, verts)
    return vo, fo


def _colmax_body(x_ref, o_ref):
    o_ref[0] = jnp.max(x_ref[0], axis=0, keepdims=True)


def _colmax(x):
    bs, v, c = x.shape
    return pl.pallas_call(
        _colmax_body,
        grid=(bs,),
        in_specs=[pl.BlockSpec((1, v, c), lambda b: (b, 0, 0))],
        out_specs=pl.BlockSpec((1, 1, c), lambda b: (b, 0, 0)),
        out_shape=jax.ShapeDtypeStruct((bs, 1, c), jnp.float32),
    )(x)


def _heads_body(fg_ref, cw1, cb1, cg, cbt, cw2, cb2,
                pw1, pb1, pg, pbt, pw2, pb2, cls_ref, hid_ref):
    fg = fg_ref[...]

    def head(w1, b1, g, bt, w2, b2):
        h = jnp.dot(fg, w1[...], preferred_element_type=jnp.float32) + b1[...]
        mu = jnp.mean(h, axis=0, keepdims=True)
        var = jnp.mean((h - mu) * (h - mu), axis=0, keepdims=True)
        h = g[...] * (h - mu) / jnp.sqrt(var + 1e-5) + bt[...]
        h = jnp.maximum(h, 0.0)
        return jnp.dot(h, w2[...], preferred_element_type=jnp.float32) + b2[...]

    cls_ref[...] = head(cw1, cb1, cg, cbt, cw2, cb2)
    hid = head(pw1, pb1, pg, pbt, pw2, pb2)
    nn = jnp.sqrt(jnp.sum(hid * hid, axis=1, keepdims=True))
    hid_ref[...] = hid / jnp.maximum(nn, 1e-12)


def _heads(fg, p):
    bs = fg.shape[0]
    args = [fg]
    for pre in ("cls", "proj"):
        args += [p[pre + "_w1"], p[pre + "_b1"].reshape(1, -1),
                 p[pre + "_bn_g"].reshape(1, -1),
                 p[pre + "_bn_b"].reshape(1, -1),
                 p[pre + "_w2"], p[pre + "_b2"].reshape(1, -1)]
    return pl.pallas_call(
        _heads_body,
        out_shape=[jax.ShapeDtypeStruct((bs, 3), jnp.float32),
                   jax.ShapeDtypeStruct((bs, 128), jnp.float32)],
    )(*args)


def kernel(vertices, params):
    p = params
    bs, v0, _ = vertices.shape

    ni1 = _knn(vertices, NEI, 256)
    fm0 = _conv(ni1, vertices, None, p["dir0"], None, 16, 256, 256)

    f1 = _mm(fm0, p["w1"], p["b1"])
    fm1 = _conv(ni1, vertices, f1[:, :, 32:], p["dir1"], f1[:, :, :32],
                32, 256, 256)

    s1 = jax.random.permutation(jax.random.key(101), v0)[: v0 // 4]
    v2, fm1p = _pool(vertices, fm1, ni1[:, :, :4], s1)

    ni2 = _knn(v2, NEI, 512)
    f2 = _mm(fm1p, p["w2"], p["b2"])
    fm2 = _conv(ni2, v2, f2[:, :, 64:], p["dir2"], f2[:, :, :64],
                64, 256, 256)

    m2 = _colmax(fm2)
    fm2c = jnp.concatenate(
        [fm2, jnp.broadcast_to(m2, (bs, 512, 64))], axis=-1)
    f3 = _mm(fm2c, p["w3"], p["b3"])
    fm3 = _conv(ni2, v2, f3[:, :, 256:], p["dir3"], f3[:, :, :256],
                256, 128, 64)

    s2 = jax.random.permutation(jax.random.key(202), 512)[:128]
    v3, fm3p = _pool(v2, fm3, ni2[:, :, :4], s2)

    ni3 = _knn(v3, NEI, 128)
    f4 = _mm(fm3p, p["w4"], p["b4"])
    fm4 = _conv(ni3, v3, f4[:, :, 256:], p["dir4"], f4[:, :, :256],
                256, 128, 64)

    m4 = _colmax(fm4)
    fm4c = jnp.concatenate(
        [fm4, jnp.broadcast_to(m4, (bs, 128, 256))], axis=-1)
    f5 = _mm(fm4c, p["w5"], p["b5"])
    fm5 = _conv(ni3, v3, f5[:, :, 1024:], p["dir5"], f5[:, :, :1024],
                1024, 64, 32)

    fg = _colmax(fm5)[:, 0, :]
    cls, hid = _heads(fg, p)
    return (fg, cls, hid)


# speedup vs baseline: 1.3142x; 1.0002x over previous
import functools
import jax
import jax.numpy as jnp
from jax import lax
from jax.experimental import pallas as pl
from jax.experimental.pallas import tpu as pltpu
from jax.experimental.pallas import tpu_sc as plsc

NEI = 32
NEG = float("-inf")
_SCNW = 32


def _sc_gather_call(B, D, CH, bpw, table, gidx):
    mesh = plsc.VectorSubcoreMesh(core_axis_name="c", subcore_axis_name="s")
    nch = bpw // CH

    @functools.partial(
        pl.kernel, mesh=mesh,
        out_type=jax.ShapeDtypeStruct((B, D), jnp.float32),
        scratch_types=[
            pltpu.VMEM((CH,), jnp.int32),
            pltpu.VMEM((CH,), jnp.int32),
            pltpu.VMEM((CH, D), jnp.float32),
            pltpu.VMEM((CH, D), jnp.float32),
            pltpu.SemaphoreType.DMA,
            pltpu.SemaphoreType.DMA,
        ],
    )
    def k(table_hbm, idx_hbm, out_hbm, ia, ib, ra, rb, sa, sb):
        wid = lax.axis_index("s") * 2 + lax.axis_index("c")
        base = wid * bpw

        def fire(iv, rv, sem, off):
            pltpu.sync_copy(idx_hbm.at[pl.ds(off, CH)], iv)
            pltpu.async_copy(table_hbm.at[iv], rv, sem)

        def drain(iv, rv, sem, off):
            pltpu.make_async_copy(table_hbm.at[iv], rv, sem).wait()
            pltpu.sync_copy(rv, out_hbm.at[pl.ds(off, CH)])

        fire(ia, ra, sa, base)

        def body(i, carry):
            off = base + 2 * i * CH
            fire(ib, rb, sb, off + CH)
            drain(ia, ra, sa, off)
            fire(ia, ra, sa, off + 2 * CH)
            drain(ib, rb, sb, off + CH)
            return carry

        lax.fori_loop(0, (nch - 2) // 2, body, 0)
        off = base + (nch - 2) * CH
        fire(ib, rb, sb, off + CH)
        drain(ia, ra, sa, off)
        drain(ib, rb, sb, off + CH)

    return k(table, gidx)


def _sc_gather(table, gidx, CH):
    R, D = table.shape
    B = gidx.shape[0]
    bpw = B // _SCNW
    return _sc_gather_call(B, D, CH, bpw, table, gidx)


def _knn_body(k, vb, v, x_ref, xr_ref, out_ref):
    X = x_ref[0]
    xr = xr_ref[0]
    inner = lax.dot_general(xr, X, (((1,), (1,)), ((), ())),
                            preferred_element_type=jnp.float32)
    qf = jnp.sum(X * X, axis=1)[None, :]
    qb = jnp.sum(xr * xr, axis=1, keepdims=True)
    neg = 2.0 * inner - qf - qb
    iota = lax.broadcasted_iota(jnp.int32, (vb, v), 1)
    for t in range(k + 1):
        j = jnp.argmax(neg, axis=1, keepdims=True).astype(jnp.int32)
        if t > 0:
            out_ref[0, :, pl.ds(t - 1, 1)] = j
        neg = jnp.where(iota == j, NEG, neg)


def _knn(vertices, k, vb):
    bs, v, _ = vertices.shape
    return pl.pallas_call(
        functools.partial(_knn_body, k, vb, v),
        grid=(bs, v // vb),
        in_specs=[pl.BlockSpec((1, v, 3), lambda b, i: (b, 0, 0)),
                  pl.BlockSpec((1, vb, 3), lambda b, i: (b, i, 0))],
        out_specs=pl.BlockSpec((1, vb, k), lambda b, i: (b, i, 0)),
        out_shape=jax.ShapeDtypeStruct((bs, v, k), jnp.int32),
    )(vertices, vertices)


def _convc_body(n, vb, c, d, has_sup,
                gg_ref, vr_ref, dir_ref, cen_ref, out_ref):
    vc = vr_ref[0]
    sd = dir_ref[...]
    nrm = jnp.sqrt(jnp.sum(sd * sd, axis=0, keepdims=True))
    sdn = sd / jnp.maximum(nrm, 1e-12)
    acc = jnp.full((vb, c), NEG, jnp.float32)
    for j in range(n):
        gj = gg_ref[0, pl.ds(j * vb, vb)]
        diff = gj[:, :3] - vc
        nn = jnp.sqrt(jnp.sum(diff * diff, axis=1, keepdims=True))
        nd = diff / jnp.maximum(nn, 1e-12)
        th = jnp.maximum(
            jnp.dot(nd, sdn, preferred_element_type=jnp.float32), 0.0)
        act = th * gj[:, 16:16 + c] if has_sup else th
        acc = jnp.maximum(acc, act)
    out_ref[0] = jnp.maximum(cen_ref[0] + acc, 0.0)


def _conv(idxs, verts, support, dirs, center, c, vb, ch):
    bs, v, n = idxs.shape
    dr = 16 + c if support is not None else 16
    d = -(-dr // 128) * 128
    pad = jnp.zeros((bs, v, 13), jnp.float32)
    parts = [verts, pad] if support is None else [verts, pad, support]
    if d > dr:
        parts.append(jnp.zeros((bs, v, d - dr), jnp.float32))
    table = jnp.concatenate(parts, axis=-1).reshape(bs * v, d)
    offs = (jnp.arange(bs, dtype=jnp.int32) * v)[:, None, None, None]
    gidx = (idxs.reshape(bs, v // vb, vb, n).transpose(0, 1, 3, 2)
            + offs).reshape(bs * n * v)
    gg = _sc_gather(table, gidx, ch).reshape(bs * (v // vb), n * vb, d)
    if center is None:
        center = jnp.zeros((bs, v, c), jnp.float32)
    nb = v // vb
    return pl.pallas_call(
        functools.partial(_convc_body, n, vb, c, d, support is not None),
        grid=(bs * nb,),
        in_specs=[pl.BlockSpec((1, n * vb, d), lambda g: (g, 0, 0)),
                  pl.BlockSpec((1, vb, 3), lambda g: (g // nb, g % nb, 0)),
                  pl.BlockSpec((3, c), lambda g: (0, 0)),
                  pl.BlockSpec((1, vb, c), lambda g: (g // nb, g % nb, 0))],
        out_specs=pl.BlockSpec((1, vb, c), lambda g: (g // nb, g % nb, 0)),
        out_shape=jax.ShapeDtypeStruct((bs, v, c), jnp.float32),
    )(gg, verts, dirs, center)


def _mm_body(x_ref, w_ref, b_ref, o_ref):
    o_ref[...] = (jnp.dot(x_ref[...], w_ref[...],
                          preferred_element_type=jnp.float32) + b_ref[...])


def _mm(x, w, b):
    bs, v, cin = x.shape
    cout = w.shape[1]
    out = pl.pallas_call(
        _mm_body,
        out_shape=jax.ShapeDtypeStruct((bs * v, cout), jnp.float32),
    )(x.reshape(bs * v, cin), w, b.reshape(1, cout))
    return out.reshape(bs, v, cout)


def _pool_body(v, np_, c, s_ref, nif_ref, fm_ref, vr_ref, fo_ref, vo_ref):
    s = s_ref[...]
    iota_pv = lax.broadcasted_iota(jnp.int32, (np_, v), 1)
    ohs = (s == iota_pv).astype(jnp.float32)
    hi = lax.Precision.HIGHEST
    niv = jnp.dot(ohs, nif_ref[0], preferred_element_type=jnp.float32,
                  precision=hi).astype(jnp.int32)
    vo_ref[0] = jnp.dot(ohs, vr_ref[0], preferred_element_type=jnp.float32,
                        precision=hi)
    acc = jnp.full((np_, c), NEG, jnp.float32)
    for j in range(4):
        oh = (iota_pv == niv[:, j:j + 1]).astype(jnp.float32)
        acc = jnp.maximum(acc, jnp.dot(oh, fm_ref[0],
                                       preferred_element_type=jnp.float32,
                                       precision=hi))
    fo_ref[0] = acc


def _pool(verts, fm, ni4, sidx):
    bs, v, c = fm.shape
    np_ = sidx.shape[0]
    fo, vo = pl.pallas_call(
        functools.partial(_pool_body, v, np_, c),
        grid=(bs,),
        in_specs=[pl.BlockSpec((np_, 1), lambda b: (0, 0)),
                  pl.BlockSpec((1, v, 4), lambda b: (b, 0, 0)),
                  pl.BlockSpec((1, v, c), lambda b: (b, 0, 0)),
                  pl.BlockSpec((1, v, 3), lambda b: (b, 0, 0))],
        out_specs=[pl.BlockSpec((1, np_, c), lambda b: (b, 0, 0)),
                   pl.BlockSpec((1, np_, 3), lambda b: (b, 0, 0))],
        out_shape=[jax.ShapeDtypeStruct((bs, np_, c), jnp.float32),
                   jax.ShapeDtypeStruct((bs, np_, 3), jnp.float32)],
    )(sidx.reshape(np_, 1).astype(jnp.int32), ni4.astype(jnp.float32),
      fm, verts)
    return vo, fo


def _colmax_body(x_ref, o_ref):
    o_ref[0] = jnp.max(x_ref[0], axis=0, keepdims=True)


def _colmax(x):
    bs, v, c = x.shape
    return pl.pallas_call(
        _colmax_body,
        grid=(bs,),
        in_specs=[pl.BlockSpec((1, v, c), lambda b: (b, 0, 0))],
        out_specs=pl.BlockSpec((1, 1, c), lambda b: (b, 0, 0)),
        out_shape=jax.ShapeDtypeStruct((bs, 1, c), jnp.float32),
    )(x)


def _heads_body(fg_ref, cw1, cb1, cg, cbt, cw2, cb2,
                pw1, pb1, pg, pbt, pw2, pb2, cls_ref, hid_ref):
    fg = fg_ref[...]

    def head(w1, b1, g, bt, w2, b2):
        h = jnp.dot(fg, w1[...], preferred_element_type=jnp.float32) + b1[...]
        mu = jnp.mean(h, axis=0, keepdims=True)
        var = jnp.mean((h - mu) * (h - mu), axis=0, keepdims=True)
        h = g[...] * (h - mu) / jnp.sqrt(var + 1e-5) + bt[...]
        h = jnp.maximum(h, 0.0)
        return jnp.dot(h, w2[...], preferred_element_type=jnp.float32) + b2[...]

    cls_ref[...] = head(cw1, cb1, cg, cbt, cw2, cb2)
    hid = head(pw1, pb1, pg, pbt, pw2, pb2)
    nn = jnp.sqrt(jnp.sum(hid * hid, axis=1, keepdims=True))
    hid_ref[...] = hid / jnp.maximum(nn, 1e-12)


def _heads(fg, p):
    bs = fg.shape[0]
    args = [fg]
    for pre in ("cls", "proj"):
        args += [p[pre + "_w1"], p[pre + "_b1"].reshape(1, -1),
                 p[pre + "_bn_g"].reshape(1, -1),
                 p[pre + "_bn_b"].reshape(1, -1),
                 p[pre + "_w2"], p[pre + "_b2"].reshape(1, -1)]
    return pl.pallas_call(
        _heads_body,
        out_shape=[jax.ShapeDtypeStruct((bs, 3), jnp.float32),
                   jax.ShapeDtypeStruct((bs, 128), jnp.float32)],
    )(*args)


def kernel(vertices, params):
    p = params
    bs, v0, _ = vertices.shape

    ni1 = _knn(vertices, NEI, 256)
    fm0 = _conv(ni1, vertices, None, p["dir0"], None, 16, 256, 256)

    f1 = _mm(fm0, p["w1"], p["b1"])
    fm1 = _conv(ni1, vertices, f1[:, :, 32:], p["dir1"], f1[:, :, :32],
                32, 256, 256)

    s1 = jax.random.permutation(jax.random.key(101), v0)[: v0 // 4]
    v2, fm1p = _pool(vertices, fm1, ni1[:, :, :4], s1)

    ni2 = _knn(v2, NEI, 512)
    f2 = _mm(fm1p, p["w2"], p["b2"])
    fm2 = _conv(ni2, v2, f2[:, :, 64:], p["dir2"], f2[:, :, :64],
                64, 256, 256)

    m2 = _colmax(fm2)
    fm2c = jnp.concatenate(
        [fm2, jnp.broadcast_to(m2, (bs, 512, 64))], axis=-1)
    f3 = _mm(fm2c, p["w3"], p["b3"])
    fm3 = _conv(ni2, v2, f3[:, :, 256:], p["dir3"], f3[:, :, :256],
                256, 128, 64)

    s2 = jax.random.permutation(jax.random.key(202), 512)[:128]
    v3, fm3p = _pool(v2, fm3, ni2[:, :, :4], s2)

    ni3 = _knn(v3, NEI, 128)
    f4 = _mm(fm3p, p["w4"], p["b4"])
    fm4 = _conv(ni3, v3, f4[:, :, 256:], p["dir4"], f4[:, :, :256],
                256, 128, 64)

    m4 = _colmax(fm4)
    fm4c = jnp.concatenate(
        [fm4, jnp.broadcast_to(m4, (bs, 128, 256))], axis=-1)
    f5 = _mm(fm4c, p["w5"], p["b5"])
    fm5 = _conv(ni3, v3, f5[:, :, 1024:], p["dir5"], f5[:, :, :1024],
                1024, 64, 32)

    fg = _colmax(fm5)[:, 0, :]
    cls, hid = _heads(fg, p)
    return (fg, cls, hid)
